# baseline (speedup 1.0000x reference)
import jax
import jax.numpy as jnp
from jax import lax
from jax.experimental import pallas as pl
from jax.experimental.pallas import tpu as pltpu
from jax.experimental.pallas import tpu_sc as plsc

VOCAB = 1_000_000
D = 64
DP = 128
BATCH = 4096
HIST = 200

NC = 2
NS = 16
NW = NC * NS
B = BATCH * HIST
B_PER_W = B // NW
CHUNK = 128
NSTEP = B_PER_W // CHUNK
NBUF = 8


def _emb_body(tok_hbm, w_hbm, out_hbm, idx_v, rows_v, gsem, ssem):
    wid = lax.axis_index("s") * NC + lax.axis_index("c")
    base = wid * B_PER_W

    pltpu.sync_copy(tok_hbm.at[pl.ds(base, B_PER_W)], idx_v)

    def gather_desc(i, b):
        return pltpu.make_async_copy(
            w_hbm.at[idx_v.at[pl.ds(i * CHUNK, CHUNK)]], rows_v.at[b], gsem)

    def scatter_desc(i, b):
        return pltpu.make_async_copy(
            rows_v.at[b],
            out_hbm.at[pl.ds(base + i * CHUNK, CHUNK), pl.ds(0, D)],
            ssem,
        )

    for b in range(NBUF):
        gather_desc(b, b).start()

    @pl.loop(0, (NSTEP - NBUF) // NBUF)
    def _steady(g):
        for b in range(NBUF):
            i = g * NBUF + b
            gather_desc(i, b).wait()
            scatter_desc(i, b).start()
            scatter_desc(i, b).wait()
            gather_desc(i + NBUF, b).start()

    for b in range(NBUF):
        i = NSTEP - NBUF + b
        gather_desc(i, b).wait()
        scatter_desc(i, b).start()
        scatter_desc(i, b).wait()


def kernel(tokens, weight):
    if tokens.dtype != jnp.int32:
        tokens = tokens.astype(jnp.int32)
    wp = jnp.concatenate(
        [weight, jnp.zeros((VOCAB, DP - D), jnp.float32)], axis=1)
    w2 = wp.reshape(2 * VOCAB, D)
    tf2 = tokens.reshape(B) * 2
    mesh = plsc.VectorSubcoreMesh(core_axis_name="c", subcore_axis_name="s")
    run = pl.kernel(
        _emb_body,
        out_type=jax.ShapeDtypeStruct((B, DP), jnp.float32),
        mesh=mesh,
        scratch_types=[
            pltpu.VMEM((B_PER_W,), jnp.int32),
            pltpu.VMEM((NBUF, CHUNK, D), jnp.float32),
            pltpu.SemaphoreType.DMA,
            pltpu.SemaphoreType.DMA,
        ],
        compiler_params=pltpu.CompilerParams(use_tc_tiling_on_sc=False),
    )
    outp = run(tf2, w2)
    return outp[:, :D].reshape(BATCH, HIST, D)
